# baseline (speedup 1.0000x reference)
import functools

import jax
import jax.numpy as jnp
from jax import lax
from jax.experimental import pallas as pl
from jax.experimental.pallas import tpu as pltpu
from jax.experimental.pallas import tpu_sc as plsc

D_IN, D_HID, D_LAT = 4096, 2048, 1024
HEADS, K, CD = 32, 256, 32
SEQ = 2048

TOK = 256
N_TILES = SEQ // TOK
KCH = D_IN // N_TILES
W2CH = D_HID // N_TILES
QROWS = 4
QTOK = SEQ // QROWS

DTOK = 128
ND_A = SEQ // DTOK
WD2CH = D_HID // ND_A

_INV_SQRT2 = 0.7071067811865476


def _gelu_exact(x):
    return 0.5 * x * (1.0 + jax.lax.erf(x * _INV_SQRT2))


def _bf(v):
    return v.astype(jnp.bfloat16)


def _dot(a, b):
    return jax.lax.dot_general(a, b, (((1,), (0,)), ((), ())),
                               preferred_element_type=jnp.float32)


def _enc_vq_kernel(x_ref, w1_ref, b1_ref, w2_ref, b2_ref,
                   cbt_ref,
                   idx_ref, fid_ref, loss_ref,
                   w1bf, w2bf, a_scr, e2_scr):
    i = pl.program_id(0)

    @pl.when(i == 0)
    def _prep():
        loss_ref[...] = jnp.zeros_like(loss_ref)
        cbt = cbt_ref[...]
        a_scr[...] = _bf(-2.0 * cbt)
        e2_scr[...] = jnp.sum(cbt * cbt, axis=1, keepdims=True)

    @pl.when(i < N_TILES)
    def _phase_a():
        w1bf[pl.ds(i * KCH, KCH), :] = _bf(w1_ref[...])
        w2bf[pl.ds(i * W2CH, W2CH), :] = _bf(w2_ref[...])

    @pl.when(i >= N_TILES)
    def _phase_b():
        t = i - N_TILES
        h = _dot(_bf(x_ref[...]), w1bf[...])
        h = _gelu_exact(h + b1_ref[...])
        z = _dot(_bf(h), w2bf[...]) + b2_ref[...]

        zbf = _bf(z)
        revk = (255.0 - jax.lax.broadcasted_iota(jnp.int32, (TOK, K), 1)
                .astype(jnp.float32))
        sum_m = jnp.zeros((TOK, 1), jnp.float32)
        idx_cols = []
        for hd in range(HEADS):
            zh_bf = zbf[:, hd * CD:(hd + 1) * CD]
            score = _dot(zh_bf, a_scr[hd]) + e2_scr[hd]
            m = jnp.min(score, axis=1, keepdims=True)
            sum_m = sum_m + m
            cand = jnp.where(score == m, revk, -1.0)
            mx = jnp.max(cand, axis=1, keepdims=True)
            idx_cols.append(255.0 - mx)
        z2row = jnp.sum(z * z, axis=1, keepdims=True)
        loss_ref[...] = loss_ref[...] + jnp.sum(z2row + sum_m).reshape(1, 1)
        idx = jnp.concatenate(idx_cols, axis=1).astype(jnp.int32)
        idx_ref[...] = idx
        hbase = jax.lax.broadcasted_iota(jnp.int32, (TOK, HEADS), 1) * K
        fid_ref[...] = idx + hbase


def _gather_codes_sc(table_pad, ids):
    NW, NCH, CHW = ids.shape
    mesh = plsc.VectorSubcoreMesh(core_axis_name="c", subcore_axis_name="s")

    @functools.partial(
        pl.kernel, mesh=mesh,
        out_type=jax.ShapeDtypeStruct((NW, NCH, CHW, 128), jnp.float32),
        scratch_types=[
            pltpu.VMEM((NCH, CHW), jnp.int32),
            pltpu.VMEM((2, CHW, 128), jnp.float32),
            pltpu.SemaphoreType.DMA,
        ],
    )
    def k(table_hbm, ids_hbm, out_hbm, idx_v, buf, gsem):
        wid = lax.axis_index("s") * 2 + lax.axis_index("c")
        pltpu.sync_copy(ids_hbm.at[wid], idx_v)
        cps = {0: pltpu.async_copy(table_hbm.at[idx_v.at[0]], buf.at[0], gsem)}
        for j in range(NCH):
            if j + 1 < NCH:
                cps[j + 1] = pltpu.async_copy(
                    table_hbm.at[idx_v.at[j + 1]], buf.at[(j + 1) % 2], gsem)
            cps[j].wait()
            pltpu.sync_copy(buf.at[j % 2], out_hbm.at[wid, j])

    return k(table_pad, ids)


def _dec_kernel(q_ref, wd1_ref, bd1_ref, wd2_ref, bd2_ref, out_ref,
                h2_all, wd2bf):
    i = pl.program_id(0)

    @pl.when(i < ND_A)
    def _phase_a():
        wd2bf[pl.ds(i * WD2CH, WD2CH), :] = _bf(wd2_ref[...])
        qp = q_ref[...]
        qt = jnp.concatenate(
            [qp[:, hd * 128:hd * 128 + CD] for hd in range(HEADS)], axis=1)
        h2 = _gelu_exact(_dot(_bf(qt), wd1_ref[...]) + bd1_ref[...])
        h2_all[pl.ds(i * DTOK, DTOK), :] = _bf(h2)

    @pl.when(i >= ND_A)
    def _phase_b():
        t = i - ND_A
        r = _dot(h2_all[pl.ds(t * TOK, TOK), :], wd2bf[...])
        out_ref[...] = r + bd2_ref[...]


def kernel(x, W_enc1, b_enc1, W_enc2, b_enc2, codebook, W_dec1, b_dec1,
           W_dec2, b_dec2):
    B, S, _ = x.shape
    x2 = x.reshape(B * S, D_IN)
    cbt_f32 = codebook.transpose(0, 2, 1)

    const = lambda *_: (0, 0)
    const3 = lambda *_: (0, 0, 0)

    idx, fid, loss = pl.pallas_call(
        _enc_vq_kernel,
        grid=(2 * N_TILES,),
        in_specs=[
            pl.BlockSpec((TOK, D_IN), lambda i: (jnp.maximum(i - N_TILES, 0), 0)),
            pl.BlockSpec((KCH, D_HID), lambda i: (jnp.minimum(i, N_TILES - 1), 0)),
            pl.BlockSpec((1, D_HID), const),
            pl.BlockSpec((W2CH, D_LAT), lambda i: (jnp.minimum(i, N_TILES - 1), 0)),
            pl.BlockSpec((1, D_LAT), const),
            pl.BlockSpec((HEADS, CD, K), const3),
        ],
        out_specs=[
            pl.BlockSpec((TOK, HEADS),
                         lambda i: (jnp.maximum(i - N_TILES, 0), 0)),
            pl.BlockSpec((TOK, HEADS),
                         lambda i: (jnp.maximum(i - N_TILES, 0), 0)),
            pl.BlockSpec((1, 1), lambda i: (0, 0)),
        ],
        out_shape=[
            jax.ShapeDtypeStruct((B * S, HEADS), jnp.int32),
            jax.ShapeDtypeStruct((B * S, HEADS), jnp.int32),
            jax.ShapeDtypeStruct((1, 1), jnp.float32),
        ],
        scratch_shapes=[
            pltpu.VMEM((D_IN, D_HID), jnp.bfloat16),
            pltpu.VMEM((D_HID, D_LAT), jnp.bfloat16),
            pltpu.VMEM((HEADS, CD, K), jnp.bfloat16),
            pltpu.VMEM((HEADS, 1, K), jnp.float32),
        ],
    )(x2, W_enc1, b_enc1.reshape(1, D_HID), W_enc2,
      b_enc2.reshape(1, D_LAT), cbt_f32)

    NW, CHW = 32, 128
    NCH = (B * S * HEADS) // (NW * CHW)
    table_pad = jnp.pad(codebook.reshape(HEADS * K, CD),
                        ((0, 0), (0, 128 - CD)))
    rows = _gather_codes_sc(table_pad, fid.reshape(NW, NCH, CHW))
    q = rows.reshape(B * S, HEADS * 128)

    recon = pl.pallas_call(
        _dec_kernel,
        grid=(ND_A + N_TILES,),
        in_specs=[
            pl.BlockSpec((DTOK, HEADS * 128),
                         lambda i: (jnp.minimum(i, ND_A - 1), 0)),
            pl.BlockSpec((D_LAT, D_HID), const),
            pl.BlockSpec((1, D_HID), const),
            pl.BlockSpec((WD2CH, D_IN), lambda i: (jnp.minimum(i, ND_A - 1), 0)),
            pl.BlockSpec((1, D_IN), const),
        ],
        out_specs=pl.BlockSpec((TOK, D_IN),
                               lambda i: (jnp.maximum(i - ND_A, 0), 0)),
        out_shape=jax.ShapeDtypeStruct((B * S, D_IN), jnp.float32),
        scratch_shapes=[
            pltpu.VMEM((SEQ, D_HID), jnp.bfloat16),
            pltpu.VMEM((D_HID, D_IN), jnp.bfloat16),
        ],
    )(q, W_dec1.astype(jnp.bfloat16), b_dec1.reshape(1, D_HID),
      W_dec2, b_dec2.reshape(1, D_IN))

    vq_loss = loss[0, 0] / (B * S * D_LAT)
    return (recon.reshape(B, S, D_IN), idx.reshape(B, S, HEADS), vq_loss)


# speedup vs baseline: 1.4064x; 1.4064x over previous
import jax
import jax.numpy as jnp
from jax.experimental import pallas as pl
from jax.experimental.pallas import tpu as pltpu

D_IN, D_HID, D_LAT = 4096, 2048, 1024
HEADS, K, CD = 32, 256, 32
SEQ = 2048

NA_E = 8
KCH = D_IN // NA_E
W2CH = D_HID // NA_E
TOK = 256
NB_E = SEQ // TOK

DTOK = 256
ND_A = SEQ // DTOK
WD2CH = D_HID // ND_A
RTOK = 256
ND_B = SEQ // RTOK

_INV_SQRT2 = 0.7071067811865476


def _gelu_exact(x):
    return 0.5 * x * (1.0 + jax.lax.erf(x * _INV_SQRT2))


def _bf(v):
    return v.astype(jnp.bfloat16)


def _dot(a, b):
    return jax.lax.dot_general(a, b, (((1,), (0,)), ((), ())),
                               preferred_element_type=jnp.float32)


def _enc_vq_kernel(x_ref, w1_ref, b1_ref, w2_ref, b2_ref,
                   cbt_ref, cbq_bf_ref,
                   q_ref, idx_ref, loss_ref,
                   w1bf, w2bf, a_scr, e2_scr):
    i = pl.program_id(0)

    @pl.when(i == 0)
    def _prep():
        loss_ref[...] = jnp.zeros_like(loss_ref)
        cbt = cbt_ref[...]
        a_scr[...] = _bf(-2.0 * cbt)
        e2_scr[...] = jnp.sum(cbt * cbt, axis=1, keepdims=True)

    @pl.when(i < NA_E)
    def _phase_a():
        w1bf[pl.ds(i * KCH, KCH), :] = _bf(w1_ref[...])
        w2bf[pl.ds(i * W2CH, W2CH), :] = _bf(w2_ref[...])

    @pl.when(i >= NA_E)
    def _phase_b():
        h = _dot(_bf(x_ref[...]), w1bf[...])
        h = _gelu_exact(h + b1_ref[...])
        z = _dot(_bf(h), w2bf[...]) + b2_ref[...]

        zbf = _bf(z)
        revk = (255.0 - jax.lax.broadcasted_iota(jnp.int32, (TOK, K), 1)
                .astype(jnp.float32))
        sum_m = jnp.zeros((TOK, 1), jnp.float32)
        idx_cols = []
        q_cols = []
        for hd in range(HEADS):
            zh_bf = zbf[:, hd * CD:(hd + 1) * CD]
            score = _dot(zh_bf, a_scr[hd]) + e2_scr[hd]
            m = jnp.min(score, axis=1, keepdims=True)
            sum_m = sum_m + m
            cand = jnp.where(score == m, revk, -1.0)
            mx = jnp.max(cand, axis=1, keepdims=True)
            onehot = _bf(cand == mx)
            q_cols.append(_dot(onehot, cbq_bf_ref[hd]))
            idx_cols.append(255.0 - mx)
        z2row = jnp.sum(z * z, axis=1, keepdims=True)
        loss_ref[...] = loss_ref[...] + jnp.sum(z2row + sum_m).reshape(1, 1)
        idx_ref[...] = jnp.concatenate(idx_cols, axis=1).astype(jnp.int32)
        q_ref[...] = _bf(jnp.concatenate(q_cols, axis=1))


def _dec_kernel(q_ref, wd1_ref, bd1_ref, wd2_ref, bd2_ref, out_ref,
                h2_all, wd2bf):
    i = pl.program_id(0)

    @pl.when(i < ND_A)
    def _phase_a():
        wd2bf[pl.ds(i * WD2CH, WD2CH), :] = _bf(wd2_ref[...])
        h2 = _gelu_exact(_dot(q_ref[...], wd1_ref[...]) + bd1_ref[...])
        h2_all[pl.ds(i * DTOK, DTOK), :] = _bf(h2)

    @pl.when(i >= ND_A)
    def _phase_b():
        t = i - ND_A
        r = _dot(h2_all[pl.ds(t * RTOK, RTOK), :], wd2bf[...])
        out_ref[...] = r + bd2_ref[...]


def kernel(x, W_enc1, b_enc1, W_enc2, b_enc2, codebook, W_dec1, b_dec1,
           W_dec2, b_dec2):
    B, S, _ = x.shape
    x2 = x.reshape(B * S, D_IN)
    cbt_f32 = codebook.transpose(0, 2, 1)
    cbq_bf = codebook.astype(jnp.bfloat16)

    const = lambda *_: (0, 0)
    const3 = lambda *_: (0, 0, 0)

    q, idx, loss = pl.pallas_call(
        _enc_vq_kernel,
        grid=(NA_E + NB_E,),
        in_specs=[
            pl.BlockSpec((TOK, D_IN), lambda i: (jnp.maximum(i - NA_E, 0), 0)),
            pl.BlockSpec((KCH, D_HID), lambda i: (jnp.minimum(i, NA_E - 1), 0)),
            pl.BlockSpec((1, D_HID), const),
            pl.BlockSpec((W2CH, D_LAT), lambda i: (jnp.minimum(i, NA_E - 1), 0)),
            pl.BlockSpec((1, D_LAT), const),
            pl.BlockSpec((HEADS, CD, K), const3),
            pl.BlockSpec((HEADS, K, CD), const3),
        ],
        out_specs=[
            pl.BlockSpec((TOK, D_LAT),
                         lambda i: (jnp.maximum(i - NA_E, 0), 0)),
            pl.BlockSpec((TOK, HEADS),
                         lambda i: (jnp.maximum(i - NA_E, 0), 0)),
            pl.BlockSpec((1, 1), lambda i: (0, 0)),
        ],
        out_shape=[
            jax.ShapeDtypeStruct((B * S, D_LAT), jnp.bfloat16),
            jax.ShapeDtypeStruct((B * S, HEADS), jnp.int32),
            jax.ShapeDtypeStruct((1, 1), jnp.float32),
        ],
        scratch_shapes=[
            pltpu.VMEM((D_IN, D_HID), jnp.bfloat16),
            pltpu.VMEM((D_HID, D_LAT), jnp.bfloat16),
            pltpu.VMEM((HEADS, CD, K), jnp.bfloat16),
            pltpu.VMEM((HEADS, 1, K), jnp.float32),
        ],
    )(x2, W_enc1, b_enc1.reshape(1, D_HID), W_enc2,
      b_enc2.reshape(1, D_LAT), cbt_f32, cbq_bf)

    recon = pl.pallas_call(
        _dec_kernel,
        grid=(ND_A + ND_B,),
        in_specs=[
            pl.BlockSpec((DTOK, D_LAT), lambda i: (jnp.minimum(i, ND_A - 1), 0)),
            pl.BlockSpec((D_LAT, D_HID), const),
            pl.BlockSpec((1, D_HID), const),
            pl.BlockSpec((WD2CH, D_IN), lambda i: (jnp.minimum(i, ND_A - 1), 0)),
            pl.BlockSpec((1, D_IN), const),
        ],
        out_specs=pl.BlockSpec((RTOK, D_IN),
                               lambda i: (jnp.maximum(i - ND_A, 0), 0)),
        out_shape=jax.ShapeDtypeStruct((B * S, D_IN), jnp.float32),
        scratch_shapes=[
            pltpu.VMEM((SEQ, D_HID), jnp.bfloat16),
            pltpu.VMEM((D_HID, D_IN), jnp.bfloat16),
        ],
    )(q, W_dec1.astype(jnp.bfloat16), b_dec1.reshape(1, D_HID),
      W_dec2, b_dec2.reshape(1, D_IN))

    vq_loss = loss[0, 0] / (B * S * D_LAT)
    return (recon.reshape(B, S, D_IN), idx.reshape(B, S, HEADS), vq_loss)
